# Initial kernel scaffold; baseline (speedup 1.0000x reference)
import jax
import jax.numpy as jnp
from jax.experimental import pallas as pl


def kernel(user_indices, item_indices, edge_index_user, edge_index_item, edge_index_user_item, emb, W_gu, att_src_gu, att_dst_gu, b_gu, W_gi, att_src_gi, att_dst_gi, b_gi, W_gui, att_src_gui, att_dst_gui, b_gui, W_mu, b_mu, W_mi, b_mi, Wq_u, bq_u, Wk_u, bk_u, Wv_u, bv_u, Wq_i, bq_i, Wk_i, bk_i, Wv_i, bv_i, W_f1, b_f1, W_f2, b_f2):
    raise NotImplementedError("write your pallas kernel here")


# speedup vs baseline: 101.1764x; 101.1764x over previous
import functools

import jax
import jax.numpy as jnp
from jax import lax
from jax.experimental import pallas as pl
from jax.experimental.pallas import tpu as pltpu
from jax.experimental.pallas import tpu_sc as plsc

NU = 25000
NI = 25000
NTOT = 50000
EMB = 32
H = 2
HC = 64
B = 4096

NC = 2
NS = 16
L = 16

NPAD = 25088
NTOTPAD = 2 * NPAD
RPT = NPAD // NS
DUMP = 25080
K = 256
KR = K // 128
KG = K // L

f32 = jnp.float32
i32 = jnp.int32

EPAD_UI = 401408
EPAD_X = 802816

_SC_PARAMS = pltpu.CompilerParams(use_tc_tiling_on_sc=False)


def _mesh():
  return plsc.VectorSubcoreMesh(
      core_axis_name="c", subcore_axis_name="s", num_cores=NC,
      num_subcores=NS)


def _prep_body(x_ref, w_ref, a8_ref, xt_ref, a8o_ref):
  xt = jnp.dot(x_ref[...], w_ref[...], preferred_element_type=f32)
  xt_ref[...] = xt
  a8o_ref[...] = lax.dot_general(a8_ref[...], xt, (((0,), (1,)), ((), ())),
                                 preferred_element_type=f32)


def _prep(x, W, A8, npad):
  n = x.shape[0]
  xp = jnp.pad(x, ((0, npad - n), (0, 0)))
  rb = 1792
  grid = (npad // rb,)
  return pl.pallas_call(
      _prep_body,
      grid=grid,
      in_specs=[
          pl.BlockSpec((rb, EMB), lambda i: (i, 0)),
          pl.BlockSpec((EMB, HC), lambda i: (0, 0)),
          pl.BlockSpec((HC, 8), lambda i: (0, 0)),
      ],
      out_specs=[
          pl.BlockSpec((rb, HC), lambda i: (i, 0)),
          pl.BlockSpec((8, rb), lambda i: (0, i)),
      ],
      out_shape=[
          jax.ShapeDtypeStruct((npad, HC), f32),
          jax.ShapeDtypeStruct((8, npad), f32),
      ],
  )(xp, W, A8)


def _att_mat(att_src, att_dst):
  eyeH = jnp.eye(H, dtype=f32)
  a_s = (att_src.reshape(H, EMB)[:, :, None] * eyeH[:, None, :]).reshape(HC, H)
  a_d = (att_dst.reshape(H, EMB)[:, :, None] * eyeH[:, None, :]).reshape(HC, H)
  return jnp.concatenate([a_s, a_d, jnp.zeros((HC, 4), f32)], axis=1)


def _leaky_exp(al):
  return jnp.exp(jnp.maximum(al, 0.2 * al))


def _edge_body(split_half, nchunks,
               src_ref, dst_ref, xt_ref, as0_ref, as1_ref, ad0_ref, ad1_ref,
               num0_ref, num1_ref, d00_ref, d01_ref, d10_ref, d11_ref,
               srcb, dstb, idxb, as0b, as1b, ad0b, ad1b,
               ex0f, ex1f, ex0b2, ex1b2, xtb,
               ntb, ia0, ia1, id0, id1, zeb0, zeb1,
               spn, spd0, spd1, sem_a, sem_x, sem_s):
  cid = lax.axis_index("c")
  sid = lax.axis_index("s")
  if split_half:
    base = jnp.asarray(0, i32)
  else:
    base = (cid * NU).astype(i32)

  r0 = sid * RPT
  if split_half:
    do_self = cid == 0
  else:
    do_self = jnp.bool_(True)

  def init_blk(j, _):
    r = r0 + j * 32
    node0 = base + r

    @pl.when(do_self)
    def _():
      pltpu.sync_copy(xt_ref.at[pl.ds(node0, 32)], ntb)
      pltpu.sync_copy(as0_ref.at[pl.ds(node0, 32)], ia0)
      pltpu.sync_copy(as1_ref.at[pl.ds(node0, 32)], ia1)
      pltpu.sync_copy(ad0_ref.at[pl.ds(node0, 32)], id0)
      pltpu.sync_copy(ad1_ref.at[pl.ds(node0, 32)], id1)
      for g in range(2):
        sl = pl.ds(g * L, L)
        zeb0[sl] = _leaky_exp(ia0[sl] + id0[sl])
        zeb1[sl] = _leaky_exp(ia1[sl] + id1[sl])
      for m in range(2):
        zv0 = zeb0[pl.ds(m * L, L)]
        zv1 = zeb1[pl.ds(m * L, L)]
        for k in range(L):
          e = m * L + k
          s0 = lax.broadcast(zv0[k], (L,))
          s1 = lax.broadcast(zv1[k], (L,))
          for k2 in range(4):
            s = s0 if k2 < 2 else s1
            ntb[e, pl.ds(k2 * L, L)] = ntb[e, pl.ds(k2 * L, L)] * s
      pltpu.sync_copy(ntb, spn.at[pl.ds(r, 32)])
      pltpu.sync_copy(zeb0, spd0.at[pl.ds(r, 32)])
      pltpu.sync_copy(zeb1, spd1.at[pl.ds(r, 32)])

    @pl.when(jnp.logical_not(do_self))
    def _():
      z = jnp.zeros((L,), f32)
      for e in range(32):
        for k2 in range(4):
          ntb[e, pl.ds(k2 * L, L)] = z
      for g in range(2):
        zeb0[pl.ds(g * L, L)] = z
        zeb1[pl.ds(g * L, L)] = z
      pltpu.sync_copy(ntb, spn.at[pl.ds(r, 32)])
      pltpu.sync_copy(zeb0, spd0.at[pl.ds(r, 32)])
      pltpu.sync_copy(zeb1, spd1.at[pl.ds(r, 32)])

    return None

  lax.fori_loop(0, RPT // 32, init_blk, None)
  plsc.subcore_barrier()

  if split_half:
    wid = cid * NS + sid
    rowstart = wid * (nchunks * KR)
  else:
    rowstart = sid * (nchunks * KR)

  def chunk(j, _):
    rb = rowstart + j * KR
    pltpu.sync_copy(src_ref.at[pl.ds(rb, KR)], srcb)
    pltpu.sync_copy(dst_ref.at[pl.ds(rb, KR)], dstb)
    pend = []
    for t in range(KR):
      sl = pl.ds(t * 128, 128)
      pend.append(pltpu.async_copy(as0_ref.at[srcb.at[t]], as0b.at[sl], sem_a))
      pend.append(pltpu.async_copy(as1_ref.at[srcb.at[t]], as1b.at[sl], sem_a))
      pend.append(pltpu.async_copy(ad0_ref.at[dstb.at[t]], ad0b.at[sl], sem_a))
      pend.append(pltpu.async_copy(ad1_ref.at[dstb.at[t]], ad1b.at[sl], sem_a))
    gx = []
    for t in range(KR):
      gx.append(pltpu.async_copy(xt_ref.at[srcb.at[t]],
                                 xtb.at[pl.ds(t * 128, 128)], sem_x))

    for t in range(KR):
      for jj in range(128 // L):
        sl = pl.ds(jj * L, L)
        loc = dstb[t, sl] - base
        ok = jnp.logical_and(loc >= 0, loc < NU)
        idxb[t, sl] = jnp.where(ok, loc, DUMP)

    for d in pend:
      d.wait()

    for g in range(KG):
      sl = pl.ds(g * L, L)
      e0 = _leaky_exp(as0b[sl] + ad0b[sl])
      e1 = _leaky_exp(as1b[sl] + ad1b[sl])
      ex0f[sl] = e0
      ex1f[sl] = e1
      ex0b2[g, pl.ds(0, L)] = e0
      ex1b2[g, pl.ds(0, L)] = e1

    for d in gx:
      d.wait()

    def heavy(m, _):
      ev0 = ex0b2[m, pl.ds(0, L)]
      ev1 = ex1b2[m, pl.ds(0, L)]
      for k in range(L):
        e = m * L + k
        s0 = lax.broadcast(ev0[k], (L,))
        s1 = lax.broadcast(ev1[k], (L,))
        for k2 in range(4):
          s = s0 if k2 < 2 else s1
          xtb[e, pl.ds(k2 * L, L)] = xtb[e, pl.ds(k2 * L, L)] * s
      return None

    lax.fori_loop(0, KG, heavy, None)

    gs = []
    for t in range(KR):
      sl = pl.ds(t * 128, 128)
      gs.append(pltpu.async_copy(xtb.at[sl], spn.at[idxb.at[t]], sem_s,
                                 add=True))
      gs.append(pltpu.async_copy(ex0f.at[sl], spd0.at[idxb.at[t]], sem_s,
                                 add=True))
      gs.append(pltpu.async_copy(ex1f.at[sl], spd1.at[idxb.at[t]], sem_s,
                                 add=True))
    for d in gs:
      d.wait()
    return None

  lax.fori_loop(0, nchunks, chunk, None)
  plsc.subcore_barrier()

  sl = pl.ds(r0, RPT)

  @pl.when(cid == 0)
  def _():
    pltpu.sync_copy(spn.at[sl], num0_ref.at[sl])
    pltpu.sync_copy(spd0.at[sl], d00_ref.at[sl])
    pltpu.sync_copy(spd1.at[sl], d01_ref.at[sl])

  @pl.when(cid == 1)
  def _():
    pltpu.sync_copy(spn.at[sl], num1_ref.at[sl])
    pltpu.sync_copy(spd0.at[sl], d10_ref.at[sl])
    pltpu.sync_copy(spd1.at[sl], d11_ref.at[sl])


def _gat_edges(src2d, dst2d, xt_tab, as0, as1, ad0, ad1, split_half, nchunks):
  body = functools.partial(_edge_body, split_half, nchunks)
  out_type = [
      jax.ShapeDtypeStruct((NPAD, HC), f32),
      jax.ShapeDtypeStruct((NPAD, HC), f32),
      jax.ShapeDtypeStruct((NPAD,), f32),
      jax.ShapeDtypeStruct((NPAD,), f32),
      jax.ShapeDtypeStruct((NPAD,), f32),
      jax.ShapeDtypeStruct((NPAD,), f32),
  ]
  scratch = [
      pltpu.VMEM((KR, 128), i32),
      pltpu.VMEM((KR, 128), i32),
      pltpu.VMEM((KR, 128), i32),
      pltpu.VMEM((K,), f32),
      pltpu.VMEM((K,), f32),
      pltpu.VMEM((K,), f32),
      pltpu.VMEM((K,), f32),
      pltpu.VMEM((K,), f32),
      pltpu.VMEM((K,), f32),
      pltpu.VMEM((KG, L), f32),
      pltpu.VMEM((KG, L), f32),
      pltpu.VMEM((K, HC), f32),
      pltpu.VMEM((32, HC), f32),
      pltpu.VMEM((32,), f32),
      pltpu.VMEM((32,), f32),
      pltpu.VMEM((32,), f32),
      pltpu.VMEM((32,), f32),
      pltpu.VMEM((32,), f32),
      pltpu.VMEM((32,), f32),
      pltpu.VMEM_SHARED((NPAD, HC), f32),
      pltpu.VMEM_SHARED((NPAD,), f32),
      pltpu.VMEM_SHARED((NPAD,), f32),
      pltpu.SemaphoreType.DMA,
      pltpu.SemaphoreType.DMA,
      pltpu.SemaphoreType.DMA,
  ]
  fn = pl.kernel(body, out_type=out_type, mesh=_mesh(),
                 scratch_types=scratch, compiler_params=_SC_PARAMS)
  return fn(src2d, dst2d, xt_tab, as0, as1, ad0, ad1)


def _gather_body(uix_ref, iix_ref,
                 nu0, nu1, u00, u01, u10, u11,
                 ni0, ni1, i00, i01, i10, i11,
                 nx0, nx1, x00, x01, x10, x11,
                 gu_ref, duA_ref, duB_ref,
                 gi_ref, diA_ref, diB_ref,
                 gxu_ref, dxuA_ref, dxuB_ref,
                 gxi_ref, dxiA_ref, dxiB_ref,
                 idxu, idxi, bufa, bufb, da, db, dc, dd, sem):

  cid = lax.axis_index("c")
  sid = lax.axis_index("s")
  wid = cid * NS + sid
  ob = wid * 128

  pltpu.sync_copy(uix_ref.at[wid], idxu)
  pltpu.sync_copy(iix_ref.at[wid], idxi)

  def pair_graph(n0, n1, A0, B0, A1, B1, g_ref, dA_ref, dB_ref, idx):
    p = [pltpu.async_copy(n0.at[idx], bufa, sem),
         pltpu.async_copy(n1.at[idx], bufb, sem),
         pltpu.async_copy(A0.at[idx], da, sem),
         pltpu.async_copy(A1.at[idx], db, sem),
         pltpu.async_copy(B0.at[idx], dc, sem),
         pltpu.async_copy(B1.at[idx], dd, sem)]
    for d in p:
      d.wait()

    def addrow(r, _):
      for k2 in range(4):
        sl = pl.ds(k2 * L, L)
        bufa[r, sl] = bufa[r, sl] + bufb[r, sl]
      return None

    lax.fori_loop(0, 128, addrow, None)
    for g in range(128 // L):
      sl = pl.ds(g * L, L)
      da[sl] = da[sl] + db[sl]
      dc[sl] = dc[sl] + dd[sl]
    pltpu.sync_copy(bufa, g_ref.at[pl.ds(ob, 128)])
    pltpu.sync_copy(da, dA_ref.at[pl.ds(ob, 128)])
    pltpu.sync_copy(dc, dB_ref.at[pl.ds(ob, 128)])

  pair_graph(nu0, nu1, u00, u01, u10, u11, gu_ref, duA_ref, duB_ref, idxu)
  pair_graph(ni0, ni1, i00, i01, i10, i11, gi_ref, diA_ref, diB_ref, idxi)

  def single_graph(n, At, Bt, g_ref, dA_ref, dB_ref, idx):
    p = [pltpu.async_copy(n.at[idx], bufa, sem),
         pltpu.async_copy(At.at[idx], da, sem),
         pltpu.async_copy(Bt.at[idx], dc, sem)]
    for d in p:
      d.wait()
    pltpu.sync_copy(bufa, g_ref.at[pl.ds(ob, 128)])
    pltpu.sync_copy(da, dA_ref.at[pl.ds(ob, 128)])
    pltpu.sync_copy(dc, dB_ref.at[pl.ds(ob, 128)])

  single_graph(nx0, x00, x01, gxu_ref, dxuA_ref, dxuB_ref, idxu)
  single_graph(nx1, x10, x11, gxi_ref, dxiA_ref, dxiB_ref, idxi)


def _batch_gather(uix2d, iix2d, u_outs, i_outs, x_outs):
  out_type = []
  for _ in range(4):
    out_type += [jax.ShapeDtypeStruct((B, HC), f32),
                 jax.ShapeDtypeStruct((B,), f32),
                 jax.ShapeDtypeStruct((B,), f32)]
  scratch = [
      pltpu.VMEM((128,), i32),
      pltpu.VMEM((128,), i32),
      pltpu.VMEM((128, HC), f32),
      pltpu.VMEM((128, HC), f32),
      pltpu.VMEM((128,), f32),
      pltpu.VMEM((128,), f32),
      pltpu.VMEM((128,), f32),
      pltpu.VMEM((128,), f32),
      pltpu.SemaphoreType.DMA,
  ]
  fn = pl.kernel(_gather_body, out_type=out_type, mesh=_mesh(),
                 scratch_types=scratch, compiler_params=_SC_PARAMS)
  return fn(uix2d, iix2d, *u_outs, *i_outs, *x_outs)


def _final_body(gu_ref, duA_ref, duB_ref, gi_ref, diA_ref, diB_ref,
                gxu_ref, dxuA_ref, dxuB_ref, gxi_ref, dxiA_ref, dxiB_ref,
                bgu_ref, bgi_ref, bgx_ref,
                wmu_ref, bmu_ref, wmi_ref, bmi_ref,
                wqu_ref, bqu_ref, wku_ref, bku_ref, wvu_ref, bvu_ref,
                wqi_ref, bqi_ref, wki_ref, bki_ref, wvi_ref, bvi_ref,
                wf1_ref, bf1_ref, wf2_ref, bf2_ref, out_ref):

  def norm(g, dA, dB, bias):
    den = jnp.concatenate(
        [jnp.broadcast_to(dA, (dA.shape[0], EMB)),
         jnp.broadcast_to(dB, (dB.shape[0], EMB))], axis=1)
    return g / (den + 1e-16) + bias

  def mm(x, w, b):
    return jnp.dot(x, w, preferred_element_type=f32) + b

  gu = norm(gu_ref[...], duA_ref[...], duB_ref[...], bgu_ref[...])
  gi = norm(gi_ref[...], diA_ref[...], diB_ref[...], bgi_ref[...])
  gxu = norm(gxu_ref[...], dxuA_ref[...], dxuB_ref[...], bgx_ref[...])
  gxi = norm(gxi_ref[...], dxiA_ref[...], dxiB_ref[...], bgx_ref[...])

  x_user = jax.nn.relu(mm(gu, wmu_ref[...], bmu_ref[...]))
  x_user_ui = jax.nn.relu(mm(gxu, wmu_ref[...], bmu_ref[...]))
  x_item = jax.nn.relu(mm(gi, wmi_ref[...], bmi_ref[...]))
  x_item_ui = jax.nn.relu(mm(gxi, wmi_ref[...], bmi_ref[...]))

  def attn2(x1, x2, wq, bq, wk, bk, wv, bv):
    Q1, Q2 = mm(x1, wq, bq), mm(x2, wq, bq)
    K1, K2 = mm(x1, wk, bk), mm(x2, wk, bk)
    V1, V2 = mm(x1, wv, bv), mm(x2, wv, bv)
    s = 1.0 / (EMB ** 0.5)
    q11 = jnp.sum(Q1 * K1, -1, keepdims=True) * s
    q12 = jnp.sum(Q1 * K2, -1, keepdims=True) * s
    q21 = jnp.sum(Q2 * K1, -1, keepdims=True) * s
    q22 = jnp.sum(Q2 * K2, -1, keepdims=True) * s

    def row(a, b):
      m = jnp.maximum(a, b)
      ea, eb = jnp.exp(a - m), jnp.exp(b - m)
      z = ea + eb
      return (ea / z) * V1 + (eb / z) * V2

    return 0.5 * (row(q11, q12) + row(q21, q22))

  att_u = attn2(x_user, x_user_ui, wqu_ref[...], bqu_ref[...],
                wku_ref[...], bku_ref[...], wvu_ref[...], bvu_ref[...])
  att_i = attn2(x_item, x_item_ui, wqi_ref[...], bqi_ref[...],
                wki_ref[...], bki_ref[...], wvi_ref[...], bvi_ref[...])
  comb = jnp.concatenate([att_u, att_i], axis=1)
  h = jax.nn.relu(mm(comb, wf1_ref[...], bf1_ref[...]))
  out_ref[...] = jax.nn.sigmoid(mm(h, wf2_ref[...], bf2_ref[...]))


def _final(args):
  return pl.pallas_call(
      _final_body,
      out_shape=jax.ShapeDtypeStruct((B, 8), f32),
  )(*args)


def _pad_edges(ei, epad, dst_pad):
  E = ei.shape[1]
  src = jnp.concatenate([ei[0].astype(i32), jnp.zeros((epad - E,), i32)])
  dst = jnp.concatenate([ei[1].astype(i32),
                         jnp.full((epad - E,), dst_pad, i32)])
  return src.reshape(-1, 128), dst.reshape(-1, 128)


def kernel(user_indices, item_indices, edge_index_user, edge_index_item,
           edge_index_user_item, emb, W_gu, att_src_gu, att_dst_gu, b_gu,
           W_gi, att_src_gi, att_dst_gi, b_gi, W_gui, att_src_gui,
           att_dst_gui, b_gui, W_mu, b_mu, W_mi, b_mi, Wq_u, bq_u, Wk_u,
           bk_u, Wv_u, bv_u, Wq_i, bq_i, Wk_i, bk_i, Wv_i, bv_i, W_f1, b_f1,
           W_f2, b_f2):
  uix = user_indices.astype(i32)
  iix = item_indices.astype(i32)

  xt_u, a8_u = _prep(emb[:NU], W_gu, _att_mat(att_src_gu, att_dst_gu), NPAD)
  xt_i, a8_i = _prep(emb[NU:], W_gi, _att_mat(att_src_gi, att_dst_gi), NPAD)
  xt_x, a8_x = _prep(emb, W_gui, _att_mat(att_src_gui, att_dst_gui), NTOTPAD)

  su, du = _pad_edges(edge_index_user, EPAD_UI, NU)
  si, di = _pad_edges(edge_index_item, EPAD_UI, NU)
  sx, dx = _pad_edges(edge_index_user_item, EPAD_X, NTOT)

  u_outs = _gat_edges(su, du, xt_u, a8_u[0], a8_u[1], a8_u[2], a8_u[3],
                      True, EPAD_UI // (NC * NS * K))
  i_outs = _gat_edges(si, di, xt_i, a8_i[0], a8_i[1], a8_i[2], a8_i[3],
                      True, EPAD_UI // (NC * NS * K))
  x_outs = _gat_edges(sx, dx, xt_x, a8_x[0], a8_x[1], a8_x[2], a8_x[3],
                      False, EPAD_X // (NS * K))

  g = _batch_gather(uix.reshape(NC * NS, 128), iix.reshape(NC * NS, 128),
                    u_outs, i_outs, x_outs)
  (gu, duA, duB, gi, diA, diB, gxu, dxuA, dxuB, gxi, dxiA, dxiB) = g

  args = [gu, duA.reshape(B, 1), duB.reshape(B, 1),
          gi, diA.reshape(B, 1), diB.reshape(B, 1),
          gxu, dxuA.reshape(B, 1), dxuB.reshape(B, 1),
          gxi, dxiA.reshape(B, 1), dxiB.reshape(B, 1),
          b_gu.reshape(1, HC), b_gi.reshape(1, HC), b_gui.reshape(1, HC),
          W_mu, b_mu.reshape(1, EMB), W_mi, b_mi.reshape(1, EMB),
          Wq_u, bq_u.reshape(1, EMB), Wk_u, bk_u.reshape(1, EMB),
          Wv_u, bv_u.reshape(1, EMB),
          Wq_i, bq_i.reshape(1, EMB), Wk_i, bk_i.reshape(1, EMB),
          Wv_i, bv_i.reshape(1, EMB),
          jnp.pad(W_f1, ((0, 0), (0, 0))), b_f1.reshape(1, EMB),
          jnp.pad(W_f2, ((0, 0), (0, 7))),
          jnp.pad(b_f2.reshape(1, 1), ((0, 0), (0, 7)))]
  out = _final(args)
  return out[:, :1]


# speedup vs baseline: 108.3890x; 1.0713x over previous
import functools

import jax
import jax.numpy as jnp
from jax import lax
from jax.experimental import pallas as pl
from jax.experimental.pallas import tpu as pltpu
from jax.experimental.pallas import tpu_sc as plsc

NU = 25000
NI = 25000
NTOT = 50000
EMB = 32
H = 2
HC = 64
B = 4096

NC = 2
NS = 16
L = 16

NPAD = 25088
NTOTPAD = 2 * NPAD
RPT = NPAD // NS
DUMP = 25080
K = 256
KR = K // 128
KG = K // L

f32 = jnp.float32
i32 = jnp.int32

EPAD_UI = 401408
EPAD_X = 802816

_SC_PARAMS = pltpu.CompilerParams(use_tc_tiling_on_sc=False)


def _mesh():
  return plsc.VectorSubcoreMesh(
      core_axis_name="c", subcore_axis_name="s", num_cores=NC,
      num_subcores=NS)


def _prep_body(x_ref, w_ref, a8_ref, xt_ref, a8o_ref):
  xt = jnp.dot(x_ref[...], w_ref[...], preferred_element_type=f32)
  xt_ref[...] = xt
  a8o_ref[...] = lax.dot_general(a8_ref[...], xt, (((0,), (1,)), ((), ())),
                                 preferred_element_type=f32)


def _prep(x, W, A8, npad):
  n = x.shape[0]
  xp = jnp.pad(x, ((0, npad - n), (0, 0)))
  rb = 1792
  grid = (npad // rb,)
  return pl.pallas_call(
      _prep_body,
      grid=grid,
      in_specs=[
          pl.BlockSpec((rb, EMB), lambda i: (i, 0)),
          pl.BlockSpec((EMB, HC), lambda i: (0, 0)),
          pl.BlockSpec((HC, 8), lambda i: (0, 0)),
      ],
      out_specs=[
          pl.BlockSpec((rb, HC), lambda i: (i, 0)),
          pl.BlockSpec((8, rb), lambda i: (0, i)),
      ],
      out_shape=[
          jax.ShapeDtypeStruct((npad, HC), f32),
          jax.ShapeDtypeStruct((8, npad), f32),
      ],
  )(xp, W, A8)


def _att_mat(att_src, att_dst):
  eyeH = jnp.eye(H, dtype=f32)
  a_s = (att_src.reshape(H, EMB)[:, :, None] * eyeH[:, None, :]).reshape(HC, H)
  a_d = (att_dst.reshape(H, EMB)[:, :, None] * eyeH[:, None, :]).reshape(HC, H)
  return jnp.concatenate([a_s, a_d, jnp.zeros((HC, 4), f32)], axis=1)


def _leaky_exp(al):
  return jnp.exp(jnp.maximum(al, 0.2 * al))


def _edge_body(split_half, npairs,
               src_ref, dst_ref, xt_ref, as0_ref, as1_ref, ad0_ref, ad1_ref,
               num0_ref, num1_ref, d00_ref, d01_ref, d10_ref, d11_ref,
               srcb0, srcb1, dstb0, dstb1, idxb0, idxb1,
               as0b0, as1b0, ad0b0, ad1b0, as0b1, as1b1, ad0b1, ad1b1,
               ex0f0, ex1f0, ex0f1, ex1f1, xtb0, xtb1,
               ntb, ia0, ia1, id0, id1, zeb0, zeb1,
               spn, spd0, spd1,
               semA0, semA1, semX0, semX1, semS0, semS1):
  cid = lax.axis_index("c")
  sid = lax.axis_index("s")
  if split_half:
    base = jnp.asarray(0, i32)
  else:
    base = (cid * NU).astype(i32)

  r0 = sid * RPT
  if split_half:
    do_self = cid == 0
  else:
    do_self = jnp.bool_(True)

  def init_blk(j, _):
    r = r0 + j * 32
    node0 = base + r

    @pl.when(do_self)
    def _():
      pltpu.sync_copy(xt_ref.at[pl.ds(node0, 32)], ntb)
      pltpu.sync_copy(as0_ref.at[pl.ds(node0, 32)], ia0)
      pltpu.sync_copy(as1_ref.at[pl.ds(node0, 32)], ia1)
      pltpu.sync_copy(ad0_ref.at[pl.ds(node0, 32)], id0)
      pltpu.sync_copy(ad1_ref.at[pl.ds(node0, 32)], id1)
      for g in range(2):
        sl = pl.ds(g * L, L)
        zeb0[sl] = _leaky_exp(ia0[sl] + id0[sl])
        zeb1[sl] = _leaky_exp(ia1[sl] + id1[sl])
      for m in range(2):
        zv0 = zeb0[pl.ds(m * L, L)]
        zv1 = zeb1[pl.ds(m * L, L)]
        for k in range(L):
          e = m * L + k
          s0 = lax.broadcast(zv0[k], (L,))
          s1 = lax.broadcast(zv1[k], (L,))
          for k2 in range(4):
            s = s0 if k2 < 2 else s1
            ntb[e, pl.ds(k2 * L, L)] = ntb[e, pl.ds(k2 * L, L)] * s
      pltpu.sync_copy(ntb, spn.at[pl.ds(r, 32)])
      pltpu.sync_copy(zeb0, spd0.at[pl.ds(r, 32)])
      pltpu.sync_copy(zeb1, spd1.at[pl.ds(r, 32)])

    @pl.when(jnp.logical_not(do_self))
    def _():
      z = jnp.zeros((L,), f32)
      for e in range(32):
        for k2 in range(4):
          ntb[e, pl.ds(k2 * L, L)] = z
      for g in range(2):
        zeb0[pl.ds(g * L, L)] = z
        zeb1[pl.ds(g * L, L)] = z
      pltpu.sync_copy(ntb, spn.at[pl.ds(r, 32)])
      pltpu.sync_copy(zeb0, spd0.at[pl.ds(r, 32)])
      pltpu.sync_copy(zeb1, spd1.at[pl.ds(r, 32)])

    return None

  lax.fori_loop(0, RPT // 32, init_blk, None)
  plsc.subcore_barrier()

  if split_half:
    wid = cid * NS + sid
    rowstart = wid * (npairs * 2)
  else:
    rowstart = sid * (npairs * 2)

  slots = [
      (srcb0, dstb0, idxb0, as0b0, as1b0, ad0b0, ad1b0, ex0f0, ex1f0, xtb0,
       semA0, semX0, semS0),
      (srcb1, dstb1, idxb1, as0b1, as1b1, ad0b1, ad1b1, ex0f1, ex1f1, xtb1,
       semA1, semX1, semS1),
  ]

  def load_rows(slot, row):
    sb, db = slots[slot][0], slots[slot][1]
    pltpu.sync_copy(src_ref.at[pl.ds(row, 1)], sb)
    pltpu.sync_copy(dst_ref.at[pl.ds(row, 1)], db)

  def fire_gathers(slot):
    (sb, db, _, a0, a1, d0, d1, _, _, xb, sA, sX, _) = slots[slot]
    pltpu.async_copy(as0_ref.at[sb.at[0]], a0.at[0], sA)
    pltpu.async_copy(as1_ref.at[sb.at[0]], a1.at[0], sA)
    pltpu.async_copy(ad0_ref.at[db.at[0]], d0.at[0], sA)
    pltpu.async_copy(ad1_ref.at[db.at[0]], d1.at[0], sA)
    pltpu.async_copy(xt_ref.at[sb.at[0]], xb, sX)

  def compute_and_scatter(slot):
    (sb, db, ib, a0, a1, d0, d1, e0f, e1f, xb, sA, sX, sS) = slots[slot]
    for _ in range(4):
      pltpu.make_async_copy(as0_ref.at[sb.at[0]], a0.at[0], sA).wait()
    for jj in range(8):
      sl2 = pl.ds(jj * L, L)
      loc = db[0, sl2] - base
      ok = jnp.logical_and(loc >= 0, loc < NU)
      ib[0, sl2] = jnp.where(ok, loc, DUMP)
      e0f[0, sl2] = _leaky_exp(a0[0, sl2] + d0[0, sl2])
      e1f[0, sl2] = _leaky_exp(a1[0, sl2] + d1[0, sl2])
    pltpu.make_async_copy(xt_ref.at[sb.at[0]], xb, sX).wait()
    for m in range(8):
      ev0 = e0f[0, pl.ds(m * L, L)]
      ev1 = e1f[0, pl.ds(m * L, L)]
      for k in range(L):
        e = m * L + k
        s0 = lax.broadcast(ev0[k], (L,))
        s1 = lax.broadcast(ev1[k], (L,))
        for k2 in range(4):
          sc = s0 if k2 < 2 else s1
          xb[e, pl.ds(k2 * L, L)] = xb[e, pl.ds(k2 * L, L)] * sc
    pltpu.async_copy(xb, spn.at[ib.at[0]], sS, add=True)
    pltpu.async_copy(e0f.at[0], spd0.at[ib.at[0]], sS, add=True)
    pltpu.async_copy(e1f.at[0], spd1.at[ib.at[0]], sS, add=True)

  def drain_scatters(slot):
    (sb, db, ib, a0, a1, d0, d1, e0f, e1f, xb, sA, sX, sS) = slots[slot]
    pltpu.make_async_copy(xb, spn.at[ib.at[0]], sS).wait()
    pltpu.make_async_copy(e0f.at[0], spd0.at[ib.at[0]], sS).wait()
    pltpu.make_async_copy(e1f.at[0], spd1.at[ib.at[0]], sS).wait()

  load_rows(0, rowstart)
  fire_gathers(0)

  def pair(s2, _):
    rb = rowstart + 2 * s2

    @pl.when(s2 > 0)
    def _():
      drain_scatters(1)

    load_rows(1, rb + 1)
    fire_gathers(1)
    compute_and_scatter(0)
    compute_and_scatter(1)

    @pl.when(s2 < npairs - 1)
    def _():
      drain_scatters(0)
      load_rows(0, rb + 2)
      fire_gathers(0)

    return None

  lax.fori_loop(0, npairs, pair, None)
  drain_scatters(0)
  drain_scatters(1)
  plsc.subcore_barrier()

  sl = pl.ds(r0, RPT)

  @pl.when(cid == 0)
  def _():
    pltpu.sync_copy(spn.at[sl], num0_ref.at[sl])
    pltpu.sync_copy(spd0.at[sl], d00_ref.at[sl])
    pltpu.sync_copy(spd1.at[sl], d01_ref.at[sl])

  @pl.when(cid == 1)
  def _():
    pltpu.sync_copy(spn.at[sl], num1_ref.at[sl])
    pltpu.sync_copy(spd0.at[sl], d10_ref.at[sl])
    pltpu.sync_copy(spd1.at[sl], d11_ref.at[sl])


def _gat_edges(src2d, dst2d, xt_tab, as0, as1, ad0, ad1, split_half, npairs):
  body = functools.partial(_edge_body, split_half, npairs)
  out_type = [
      jax.ShapeDtypeStruct((NPAD, HC), f32),
      jax.ShapeDtypeStruct((NPAD, HC), f32),
      jax.ShapeDtypeStruct((NPAD,), f32),
      jax.ShapeDtypeStruct((NPAD,), f32),
      jax.ShapeDtypeStruct((NPAD,), f32),
      jax.ShapeDtypeStruct((NPAD,), f32),
  ]
  scratch = (
      [pltpu.VMEM((1, 128), i32) for _ in range(6)] +
      [pltpu.VMEM((1, 128), f32) for _ in range(8)] +
      [pltpu.VMEM((1, 128), f32) for _ in range(4)] +
      [pltpu.VMEM((128, HC), f32) for _ in range(2)] +
      [pltpu.VMEM((32, HC), f32),
       pltpu.VMEM((32,), f32), pltpu.VMEM((32,), f32),
       pltpu.VMEM((32,), f32), pltpu.VMEM((32,), f32),
       pltpu.VMEM((32,), f32), pltpu.VMEM((32,), f32),
       pltpu.VMEM_SHARED((NPAD, HC), f32),
       pltpu.VMEM_SHARED((NPAD,), f32),
       pltpu.VMEM_SHARED((NPAD,), f32)] +
      [pltpu.SemaphoreType.DMA for _ in range(6)]
  )
  fn = pl.kernel(body, out_type=out_type, mesh=_mesh(),
                 scratch_types=scratch, compiler_params=_SC_PARAMS)
  return fn(src2d, dst2d, xt_tab, as0, as1, ad0, ad1)


def _gather_body(uix_ref, iix_ref,
                 nu0, nu1, u00, u01, u10, u11,
                 ni0, ni1, i00, i01, i10, i11,
                 nx0, nx1, x00, x01, x10, x11,
                 gu_ref, duA_ref, duB_ref,
                 gi_ref, diA_ref, diB_ref,
                 gxu_ref, dxuA_ref, dxuB_ref,
                 gxi_ref, dxiA_ref, dxiB_ref,
                 idxu, idxi, bufa, bufb, da, db, dc, dd, sem):

  cid = lax.axis_index("c")
  sid = lax.axis_index("s")
  wid = cid * NS + sid
  ob = wid * 128

  pltpu.sync_copy(uix_ref.at[wid], idxu)
  pltpu.sync_copy(iix_ref.at[wid], idxi)

  def pair_graph(n0, n1, A0, B0, A1, B1, g_ref, dA_ref, dB_ref, idx):
    p = [pltpu.async_copy(n0.at[idx], bufa, sem),
         pltpu.async_copy(n1.at[idx], bufb, sem),
         pltpu.async_copy(A0.at[idx], da, sem),
         pltpu.async_copy(A1.at[idx], db, sem),
         pltpu.async_copy(B0.at[idx], dc, sem),
         pltpu.async_copy(B1.at[idx], dd, sem)]
    for d in p:
      d.wait()

    def addrow(r, _):
      for k2 in range(4):
        sl = pl.ds(k2 * L, L)
        bufa[r, sl] = bufa[r, sl] + bufb[r, sl]
      return None

    lax.fori_loop(0, 128, addrow, None)
    for g in range(128 // L):
      sl = pl.ds(g * L, L)
      da[sl] = da[sl] + db[sl]
      dc[sl] = dc[sl] + dd[sl]
    pltpu.sync_copy(bufa, g_ref.at[pl.ds(ob, 128)])
    pltpu.sync_copy(da, dA_ref.at[pl.ds(ob, 128)])
    pltpu.sync_copy(dc, dB_ref.at[pl.ds(ob, 128)])

  pair_graph(nu0, nu1, u00, u01, u10, u11, gu_ref, duA_ref, duB_ref, idxu)
  pair_graph(ni0, ni1, i00, i01, i10, i11, gi_ref, diA_ref, diB_ref, idxi)

  def single_graph(n, At, Bt, g_ref, dA_ref, dB_ref, idx):
    p = [pltpu.async_copy(n.at[idx], bufa, sem),
         pltpu.async_copy(At.at[idx], da, sem),
         pltpu.async_copy(Bt.at[idx], dc, sem)]
    for d in p:
      d.wait()
    pltpu.sync_copy(bufa, g_ref.at[pl.ds(ob, 128)])
    pltpu.sync_copy(da, dA_ref.at[pl.ds(ob, 128)])
    pltpu.sync_copy(dc, dB_ref.at[pl.ds(ob, 128)])

  single_graph(nx0, x00, x01, gxu_ref, dxuA_ref, dxuB_ref, idxu)
  single_graph(nx1, x10, x11, gxi_ref, dxiA_ref, dxiB_ref, idxi)


def _batch_gather(uix2d, iix2d, u_outs, i_outs, x_outs):
  out_type = []
  for _ in range(4):
    out_type += [jax.ShapeDtypeStruct((B, HC), f32),
                 jax.ShapeDtypeStruct((B,), f32),
                 jax.ShapeDtypeStruct((B,), f32)]
  scratch = [
      pltpu.VMEM((128,), i32),
      pltpu.VMEM((128,), i32),
      pltpu.VMEM((128, HC), f32),
      pltpu.VMEM((128, HC), f32),
      pltpu.VMEM((128,), f32),
      pltpu.VMEM((128,), f32),
      pltpu.VMEM((128,), f32),
      pltpu.VMEM((128,), f32),
      pltpu.SemaphoreType.DMA,
  ]
  fn = pl.kernel(_gather_body, out_type=out_type, mesh=_mesh(),
                 scratch_types=scratch, compiler_params=_SC_PARAMS)
  return fn(uix2d, iix2d, *u_outs, *i_outs, *x_outs)


def _final_body(gu_ref, duA_ref, duB_ref, gi_ref, diA_ref, diB_ref,
                gxu_ref, dxuA_ref, dxuB_ref, gxi_ref, dxiA_ref, dxiB_ref,
                bgu_ref, bgi_ref, bgx_ref,
                wmu_ref, bmu_ref, wmi_ref, bmi_ref,
                wqu_ref, bqu_ref, wku_ref, bku_ref, wvu_ref, bvu_ref,
                wqi_ref, bqi_ref, wki_ref, bki_ref, wvi_ref, bvi_ref,
                wf1_ref, bf1_ref, wf2_ref, bf2_ref, out_ref):

  def norm(g, dA, dB, bias):
    den = jnp.concatenate(
        [jnp.broadcast_to(dA, (dA.shape[0], EMB)),
         jnp.broadcast_to(dB, (dB.shape[0], EMB))], axis=1)
    return g / (den + 1e-16) + bias

  def mm(x, w, b):
    return jnp.dot(x, w, preferred_element_type=f32) + b

  gu = norm(gu_ref[...], duA_ref[...], duB_ref[...], bgu_ref[...])
  gi = norm(gi_ref[...], diA_ref[...], diB_ref[...], bgi_ref[...])
  gxu = norm(gxu_ref[...], dxuA_ref[...], dxuB_ref[...], bgx_ref[...])
  gxi = norm(gxi_ref[...], dxiA_ref[...], dxiB_ref[...], bgx_ref[...])

  x_user = jax.nn.relu(mm(gu, wmu_ref[...], bmu_ref[...]))
  x_user_ui = jax.nn.relu(mm(gxu, wmu_ref[...], bmu_ref[...]))
  x_item = jax.nn.relu(mm(gi, wmi_ref[...], bmi_ref[...]))
  x_item_ui = jax.nn.relu(mm(gxi, wmi_ref[...], bmi_ref[...]))

  def attn2(x1, x2, wq, bq, wk, bk, wv, bv):
    Q1, Q2 = mm(x1, wq, bq), mm(x2, wq, bq)
    K1, K2 = mm(x1, wk, bk), mm(x2, wk, bk)
    V1, V2 = mm(x1, wv, bv), mm(x2, wv, bv)
    s = 1.0 / (EMB ** 0.5)
    q11 = jnp.sum(Q1 * K1, -1, keepdims=True) * s
    q12 = jnp.sum(Q1 * K2, -1, keepdims=True) * s
    q21 = jnp.sum(Q2 * K1, -1, keepdims=True) * s
    q22 = jnp.sum(Q2 * K2, -1, keepdims=True) * s

    def row(a, b):
      m = jnp.maximum(a, b)
      ea, eb = jnp.exp(a - m), jnp.exp(b - m)
      z = ea + eb
      return (ea / z) * V1 + (eb / z) * V2

    return 0.5 * (row(q11, q12) + row(q21, q22))

  att_u = attn2(x_user, x_user_ui, wqu_ref[...], bqu_ref[...],
                wku_ref[...], bku_ref[...], wvu_ref[...], bvu_ref[...])
  att_i = attn2(x_item, x_item_ui, wqi_ref[...], bqi_ref[...],
                wki_ref[...], bki_ref[...], wvi_ref[...], bvi_ref[...])
  comb = jnp.concatenate([att_u, att_i], axis=1)
  h = jax.nn.relu(mm(comb, wf1_ref[...], bf1_ref[...]))
  out_ref[...] = jax.nn.sigmoid(mm(h, wf2_ref[...], bf2_ref[...]))


def _final(args):
  return pl.pallas_call(
      _final_body,
      out_shape=jax.ShapeDtypeStruct((B, 8), f32),
  )(*args)


def _pad_edges(ei, epad, dst_pad):
  E = ei.shape[1]
  src = jnp.concatenate([ei[0].astype(i32), jnp.zeros((epad - E,), i32)])
  dst = jnp.concatenate([ei[1].astype(i32),
                         jnp.full((epad - E,), dst_pad, i32)])
  return src.reshape(-1, 128), dst.reshape(-1, 128)


def kernel(user_indices, item_indices, edge_index_user, edge_index_item,
           edge_index_user_item, emb, W_gu, att_src_gu, att_dst_gu, b_gu,
           W_gi, att_src_gi, att_dst_gi, b_gi, W_gui, att_src_gui,
           att_dst_gui, b_gui, W_mu, b_mu, W_mi, b_mi, Wq_u, bq_u, Wk_u,
           bk_u, Wv_u, bv_u, Wq_i, bq_i, Wk_i, bk_i, Wv_i, bv_i, W_f1, b_f1,
           W_f2, b_f2):
  uix = user_indices.astype(i32)
  iix = item_indices.astype(i32)

  xt_u, a8_u = _prep(emb[:NU], W_gu, _att_mat(att_src_gu, att_dst_gu), NPAD)
  xt_i, a8_i = _prep(emb[NU:], W_gi, _att_mat(att_src_gi, att_dst_gi), NPAD)
  xt_x, a8_x = _prep(emb, W_gui, _att_mat(att_src_gui, att_dst_gui), NTOTPAD)

  su, du = _pad_edges(edge_index_user, EPAD_UI, NU)
  si, di = _pad_edges(edge_index_item, EPAD_UI, NU)
  sx, dx = _pad_edges(edge_index_user_item, EPAD_X, NTOT)

  u_outs = _gat_edges(su, du, xt_u, a8_u[0], a8_u[1], a8_u[2], a8_u[3],
                      True, EPAD_UI // (NC * NS * 256))
  i_outs = _gat_edges(si, di, xt_i, a8_i[0], a8_i[1], a8_i[2], a8_i[3],
                      True, EPAD_UI // (NC * NS * 256))
  x_outs = _gat_edges(sx, dx, xt_x, a8_x[0], a8_x[1], a8_x[2], a8_x[3],
                      False, EPAD_X // (NS * 256))

  g = _batch_gather(uix.reshape(NC * NS, 128), iix.reshape(NC * NS, 128),
                    u_outs, i_outs, x_outs)
  (gu, duA, duB, gi, diA, diB, gxu, dxuA, dxuB, gxi, dxiA, dxiB) = g

  args = [gu, duA.reshape(B, 1), duB.reshape(B, 1),
          gi, diA.reshape(B, 1), diB.reshape(B, 1),
          gxu, dxuA.reshape(B, 1), dxuB.reshape(B, 1),
          gxi, dxiA.reshape(B, 1), dxiB.reshape(B, 1),
          b_gu.reshape(1, HC), b_gi.reshape(1, HC), b_gui.reshape(1, HC),
          W_mu, b_mu.reshape(1, EMB), W_mi, b_mi.reshape(1, EMB),
          Wq_u, bq_u.reshape(1, EMB), Wk_u, bk_u.reshape(1, EMB),
          Wv_u, bv_u.reshape(1, EMB),
          Wq_i, bq_i.reshape(1, EMB), Wk_i, bk_i.reshape(1, EMB),
          Wv_i, bv_i.reshape(1, EMB),
          jnp.pad(W_f1, ((0, 0), (0, 0))), b_f1.reshape(1, EMB),
          jnp.pad(W_f2, ((0, 0), (0, 7))),
          jnp.pad(b_f2.reshape(1, 1), ((0, 0), (0, 7)))]
  out = _final(args)
  return out[:, :1]
